# baseline (speedup 1.0000x reference)
import jax
import jax.numpy as jnp
from jax.experimental import pallas as pl
from jax.experimental.pallas import tpu as pltpu

_NSEG = 64
_NEG = -1e30


def _pool_kernel(x_ref, b_ref, wg_ref, bg_ref, wn_ref, bn_ref, out_ref,
                 m_ref, s_ref, acc_ref):
    i = pl.program_id(0)
    nb = pl.num_programs(0)

    @pl.when(i == 0)
    def _init():
        m_ref[:] = jnp.full_like(m_ref, _NEG)
        s_ref[:] = jnp.zeros_like(s_ref)
        acc_ref[:] = jnp.zeros_like(acc_ref)

    blk = x_ref.shape[0]
    nsplit = 5
    step = blk // nsplit

    def _stage(lo, hi):
        xb = x_ref[lo:hi].astype(jnp.bfloat16)
        G = jnp.dot(xb, wg_ref[:], preferred_element_type=jnp.float32)
        G = G + bg_ref[0, 0]
        b = jnp.transpose(b_ref[0, :, lo:hi])
        seg = jax.lax.broadcasted_iota(jnp.int32, (hi - lo, _NSEG), 1)
        mask = b == seg
        masked_G = jnp.where(mask, G, -3e38)
        bmax = jnp.max(masked_G, axis=0, keepdims=True)
        return xb, masked_G, bmax

    parts = [_stage(k * step, (k + 1) * step) for k in range(nsplit)]

    m_old = m_ref[:]
    bmax = parts[0][2]
    for k in range(1, nsplit):
        bmax = jnp.maximum(bmax, parts[k][2])
    m_new = jnp.maximum(m_old, bmax)
    alpha = jnp.exp(m_old - m_new)

    seg_e = jnp.zeros_like(m_new)
    acc_upd = None
    for xb, mG, _ in parts:
        E = jnp.exp(mG - m_new)
        seg_e = seg_e + jnp.sum(E, axis=0, keepdims=True)
        Et = jnp.transpose(E.astype(jnp.bfloat16))
        d_upd = jnp.dot(Et, xb, preferred_element_type=jnp.float32)
        acc_upd = d_upd if acc_upd is None else acc_upd + d_upd

    m_ref[:] = m_new
    s_ref[:] = alpha * s_ref[:] + seg_e
    acc_ref[:] = jnp.reshape(alpha, (_NSEG, 1)) * acc_ref[:] + acc_upd

    @pl.when(i == nb - 1)
    def _finish():
        s = s_ref[:]
        scale = 1.0 / (s + 1e-16)
        pooled = acc_ref[:] * jnp.reshape(scale, (_NSEG, 1))
        out = jnp.dot(pooled, wn_ref[:], preferred_element_type=jnp.float32)
        frac = jnp.reshape(s * scale, (_NSEG, 1))
        out_ref[:] = out + frac * bn_ref[:]


def _pick_block(n):
    for blk in (5000, 2000, 1000, 500, 200, 100, 8):
        if n % blk == 0:
            return blk
    return n


def kernel(x, Wg, bg, Wn, bn, batch, size):
    n, d = x.shape
    blk = _pick_block(n)
    grid = n // blk
    b2 = batch.astype(jnp.int32).reshape(grid, 1, blk)
    wg_rep = jnp.broadcast_to(Wg, (d, _NSEG)).astype(jnp.bfloat16)
    out = pl.pallas_call(
        _pool_kernel,
        grid=(grid,),
        in_specs=[
            pl.BlockSpec((blk, d), lambda i: (i, 0)),
            pl.BlockSpec((1, 1, blk), lambda i: (i, 0, 0)),
            pl.BlockSpec((d, _NSEG), lambda i: (0, 0)),
            pl.BlockSpec((1, 1), lambda i: (0, 0)),
            pl.BlockSpec((d, d), lambda i: (0, 0)),
            pl.BlockSpec((1, d), lambda i: (0, 0)),
        ],
        out_specs=pl.BlockSpec((_NSEG, d), lambda i: (0, 0)),
        out_shape=jax.ShapeDtypeStruct((_NSEG, d), jnp.float32),
        scratch_shapes=[
            pltpu.VMEM((1, _NSEG), jnp.float32),
            pltpu.VMEM((1, _NSEG), jnp.float32),
            pltpu.VMEM((_NSEG, d), jnp.float32),
        ],
    )(x, b2, wg_rep, bg.reshape(1, 1), Wn, bn.reshape(1, d))
    return out


# speedup vs baseline: 1.1879x; 1.1879x over previous
import jax
import jax.numpy as jnp
from jax.experimental import pallas as pl
from jax.experimental.pallas import tpu as pltpu

_NSEG = 64
_NEG = -1e30


def _pool_kernel(x1_ref, x2_ref, b1_ref, b2_ref, wg_ref, wn_ref, bn_ref,
                 out_ref, m_ref, s_ref, acc_ref):
    i = pl.program_id(0)
    nb = pl.num_programs(0)

    @pl.when(i == 0)
    def _init():
        m_ref[:] = jnp.full_like(m_ref, _NEG)
        s_ref[:] = jnp.zeros_like(s_ref)
        acc_ref[:] = jnp.zeros_like(acc_ref)

    def _stage(x_ref, b_ref):
        xb = x_ref[:].astype(jnp.bfloat16)
        G = jnp.dot(xb, wg_ref[:], preferred_element_type=jnp.float32)
        b = jnp.transpose(b_ref[0])
        seg = jax.lax.broadcasted_iota(jnp.int32, (b.shape[0], _NSEG), 1)
        mask = b == seg
        masked_G = jnp.where(mask, G, -3e38)
        bmax = jnp.max(masked_G, axis=0, keepdims=True)
        return xb, masked_G, bmax

    xb1, mG1, bmax1 = _stage(x1_ref, b1_ref)
    xb2, mG2, bmax2 = _stage(x2_ref, b2_ref)

    m_old = m_ref[:]
    m_new = jnp.maximum(m_old, jnp.maximum(bmax1, bmax2))
    alpha = jnp.exp(m_old - m_new)

    E1 = jnp.exp(mG1 - m_new)
    E2 = jnp.exp(mG2 - m_new)
    seg_e = (jnp.sum(E1, axis=0, keepdims=True)
             + jnp.sum(E2, axis=0, keepdims=True))
    Et1 = jnp.transpose(E1.astype(jnp.bfloat16))
    Et2 = jnp.transpose(E2.astype(jnp.bfloat16))
    acc_upd = (jnp.dot(Et1, xb1, preferred_element_type=jnp.float32)
               + jnp.dot(Et2, xb2, preferred_element_type=jnp.float32))

    m_ref[:] = m_new
    s_ref[:] = alpha * s_ref[:] + seg_e
    acc_ref[:] = jnp.reshape(alpha, (_NSEG, 1)) * acc_ref[:] + acc_upd

    @pl.when(i == nb - 1)
    def _finish():
        s = s_ref[:]
        scale = 1.0 / (s + 1e-16)
        pooled = acc_ref[:] * jnp.reshape(scale, (_NSEG, 1))
        out = jnp.dot(pooled, wn_ref[:], preferred_element_type=jnp.float32)
        frac = jnp.reshape(s * scale, (_NSEG, 1))
        out_ref[:] = out + frac * bn_ref[:]


def _pick_block(n):
    for blk in (5000, 2500, 1000, 500, 100, 20, 4, 1):
        if n % (2 * blk) == 0:
            return blk
    return n


def kernel(x, Wg, bg, Wn, bn, batch, size):
    n, d = x.shape
    blk = _pick_block(n)
    grid = n // (2 * blk)
    b3 = batch.astype(jnp.int32).reshape(2 * grid, 1, blk)
    wg_rep = jnp.broadcast_to(Wg, (d, _NSEG)).astype(jnp.bfloat16)
    out = pl.pallas_call(
        _pool_kernel,
        grid=(grid,),
        in_specs=[
            pl.BlockSpec((blk, d), lambda i: (i, 0)),
            pl.BlockSpec((blk, d), lambda i: (i + pl.num_programs(0), 0)),
            pl.BlockSpec((1, 1, blk), lambda i: (i, 0, 0)),
            pl.BlockSpec((1, 1, blk), lambda i: (i + pl.num_programs(0), 0, 0)),
            pl.BlockSpec((d, _NSEG), lambda i: (0, 0)),
            pl.BlockSpec((d, d), lambda i: (0, 0)),
            pl.BlockSpec((1, d), lambda i: (0, 0)),
        ],
        out_specs=pl.BlockSpec((_NSEG, d), lambda i: (0, 0)),
        out_shape=jax.ShapeDtypeStruct((_NSEG, d), jnp.float32),
        scratch_shapes=[
            pltpu.VMEM((1, _NSEG), jnp.float32),
            pltpu.VMEM((1, _NSEG), jnp.float32),
            pltpu.VMEM((_NSEG, d), jnp.float32),
        ],
    )(x, x, b3, b3, wg_rep, Wn, bn.reshape(1, d))
    return out


# speedup vs baseline: 1.2352x; 1.0398x over previous
import jax
import jax.numpy as jnp
from jax.experimental import pallas as pl
from jax.experimental.pallas import tpu as pltpu

_NSEG = 64
_NEG = -1e30


def _pool_kernel(x_ref, b_ref, wg_ref, wn_ref, bn_ref,
                 out_ref, m_ref, s_ref, acc_ref):
    i = pl.program_id(0)
    nb = pl.num_programs(0)

    @pl.when(i == 0)
    def _init():
        m_ref[:] = jnp.full_like(m_ref, _NEG)
        s_ref[:] = jnp.zeros_like(s_ref)
        acc_ref[:] = jnp.zeros_like(acc_ref)

    blk = x_ref.shape[0]
    half = blk // 2

    def _stage(lo, hi):
        xb = x_ref[lo:hi].astype(jnp.bfloat16)
        G = jnp.dot(xb, wg_ref[:], preferred_element_type=jnp.float32)
        b = jnp.transpose(b_ref[0, :, lo:hi])
        seg = jax.lax.broadcasted_iota(jnp.int32, (hi - lo, _NSEG), 1)
        mask = b == seg
        masked_G = jnp.where(mask, G, -3e38)
        bmax = jnp.max(masked_G, axis=0, keepdims=True)
        return xb, masked_G, bmax

    xb1, mG1, bmax1 = _stage(0, half)
    xb2, mG2, bmax2 = _stage(half, blk)

    m_old = m_ref[:]
    m_new = jnp.maximum(m_old, jnp.maximum(bmax1, bmax2))
    alpha = jnp.exp(m_old - m_new)

    E1 = jnp.exp(mG1 - m_new)
    E2 = jnp.exp(mG2 - m_new)
    seg_e = (jnp.sum(E1, axis=0, keepdims=True)
             + jnp.sum(E2, axis=0, keepdims=True))
    Et1 = jnp.transpose(E1.astype(jnp.bfloat16))
    Et2 = jnp.transpose(E2.astype(jnp.bfloat16))
    acc_upd = (jnp.dot(Et1, xb1, preferred_element_type=jnp.float32)
               + jnp.dot(Et2, xb2, preferred_element_type=jnp.float32))

    m_ref[:] = m_new
    s_ref[:] = alpha * s_ref[:] + seg_e
    acc_ref[:] = jnp.reshape(alpha, (_NSEG, 1)) * acc_ref[:] + acc_upd

    @pl.when(i == nb - 1)
    def _finish():
        s = s_ref[:]
        scale = 1.0 / (s + 1e-16)
        pooled = acc_ref[:] * jnp.reshape(scale, (_NSEG, 1))
        out = jnp.dot(pooled, wn_ref[:], preferred_element_type=jnp.float32)
        frac = jnp.reshape(s * scale, (_NSEG, 1))
        out_ref[:] = out + frac * bn_ref[:]


def _pick_block(n):
    for blk in (5000, 2000, 1000, 500, 200, 100, 8):
        if n % blk == 0:
            return blk
    return n


def kernel(x, Wg, bg, Wn, bn, batch, size):
    n, d = x.shape
    blk = _pick_block(n)
    grid = n // blk
    b3 = batch.astype(jnp.int32).reshape(grid, 1, blk)
    wg_rep = jnp.broadcast_to(Wg, (d, _NSEG)).astype(jnp.bfloat16)
    out = pl.pallas_call(
        _pool_kernel,
        grid=(grid,),
        in_specs=[
            pl.BlockSpec((blk, d), lambda i: (i, 0)),
            pl.BlockSpec((1, 1, blk), lambda i: (i, 0, 0)),
            pl.BlockSpec((d, _NSEG), lambda i: (0, 0)),
            pl.BlockSpec((d, d), lambda i: (0, 0)),
            pl.BlockSpec((1, d), lambda i: (0, 0)),
        ],
        out_specs=pl.BlockSpec((_NSEG, d), lambda i: (0, 0)),
        out_shape=jax.ShapeDtypeStruct((_NSEG, d), jnp.float32),
        scratch_shapes=[
            pltpu.VMEM((1, _NSEG), jnp.float32),
            pltpu.VMEM((1, _NSEG), jnp.float32),
            pltpu.VMEM((_NSEG, d), jnp.float32),
        ],
    )(x, b3, wg_rep, Wn, bn.reshape(1, d))
    return out
